# Initial kernel scaffold; baseline (speedup 1.0000x reference)
import jax
import jax.numpy as jnp
from jax.experimental import pallas as pl


def kernel(edge_index, adj_values, embeds, W1, b1, W2, b2, Wa, ba):
    raise NotImplementedError("write your pallas kernel here")


# speedup vs baseline: 11.5252x; 11.5252x over previous
import functools

import jax
import jax.numpy as jnp
from jax import lax
from jax.experimental import pallas as pl
from jax.experimental.pallas import tpu as pltpu
from jax.experimental.pallas import tpu_sc as plsc

N = 10000
E = 320000
H = 128
NC = 2
NS = 16
L = 16
NW = NC * NS
EW = E // NW
C = 80
NCH = EW // C
NP2 = 10240
RPT = NP2 // NS
NPAD = 640


def _scores_body(emb_ref, w1_ref, b1_ref, w2_ref, b2_ref, wa1_ref, wa2_ref,
                 ba_ref, a1_ref, a2_ref):
    emb = emb_ref[:, :]
    h1 = jnp.maximum(
        jnp.dot(emb, w1_ref[:, :], preferred_element_type=jnp.float32,
                precision=lax.Precision.HIGHEST) + b1_ref[:, :], 0.0)
    h2 = jnp.maximum(
        jnp.dot(emb, w2_ref[:, :], preferred_element_type=jnp.float32,
                precision=lax.Precision.HIGHEST) + b2_ref[:, :], 0.0)
    a1_ref[:, :] = jnp.sum(h1 * wa1_ref[:, :], axis=1, keepdims=True) + ba_ref[0, 0]
    a2_ref[:, :] = jnp.sum(h2 * wa2_ref[:, :], axis=1, keepdims=True)


def _node_scores(embeds, W1, b1, W2, b2, wa1, wa2, ba):
    blk = 2000
    grid = N // blk
    a1, a2 = pl.pallas_call(
        _scores_body,
        grid=(grid,),
        in_specs=[
            pl.BlockSpec((blk, H), lambda i: (i, 0)),
            pl.BlockSpec((H, H), lambda i: (0, 0)),
            pl.BlockSpec((1, H), lambda i: (0, 0)),
            pl.BlockSpec((H, H), lambda i: (0, 0)),
            pl.BlockSpec((1, H), lambda i: (0, 0)),
            pl.BlockSpec((1, H), lambda i: (0, 0)),
            pl.BlockSpec((1, H), lambda i: (0, 0)),
            pl.BlockSpec(memory_space=pltpu.SMEM),
        ],
        out_specs=[
            pl.BlockSpec((blk, 1), lambda i: (i, 0)),
            pl.BlockSpec((blk, 1), lambda i: (i, 0)),
        ],
        out_shape=[
            jax.ShapeDtypeStruct((N, 1), jnp.float32),
            jax.ShapeDtypeStruct((N, 1), jnp.float32),
        ],
    )(embeds, W1, b1.reshape(1, H), W2, b2.reshape(1, H), wa1, wa2,
      ba.reshape(1, 1))
    return a1.reshape(N), a2.reshape(N)


KC = 5
CB = KC * C
NCB = EW // CB
RS_R = 80


def _sc_body(row4, col4, adj4, a1_hbm, a2_hbm, emb_hbm,
             values_hbm, partial_hbm,
             a1b, a2b, rs_loc, row_cb, col_cb, adj_cb,
             vals_cb, rows_buf, idx2,
             rowsum_sh, out_sh):
    s = lax.axis_index("s")
    c = lax.axis_index("c")
    w_own = c * NS + s
    w_mir = (1 - c) * NS + s

    pltpu.sync_copy(a1_hbm, a1b)
    pltpu.sync_copy(a2_hbm, a2b)

    zeros16 = jnp.zeros((L,), jnp.float32)

    @pl.loop(0, RS_R)
    def _(i):
        for j in range(128 // L):
            rs_loc[i, pl.ds(j * L, L)] = zeros16

    for k in range(RS_R // L):
        idx2[0, pl.ds(k * L, L)] = lax.iota(jnp.int32, L) + k * L

    def _split(r):
        return [lax.shift_right_logical(r, 7), lax.bitwise_and(r, 127)]

    @pl.loop(0, 2)
    def _(m):
        w = jnp.where(m == 0, w_own, w_mir)

        @pl.loop(0, NCB)
        def _(cb):
            pltpu.sync_copy(row4.at[w, cb], row_cb)
            pltpu.sync_copy(col4.at[w, cb], col_cb)

            @pl.loop(0, KC)
            def _(k):
                for j in range(C // L):
                    r = row_cb[k, pl.ds(j * L, L)]
                    cc = col_cb[k, pl.ds(j * L, L)]
                    e = jnp.exp(plsc.load_gather(a1b, [r]) +
                                plsc.load_gather(a2b, [cc]))
                    plsc.addupdate_scatter(rs_loc, _split(r), e)

    @pl.when(s == 0)
    def _():
        pltpu.sync_copy(rs_loc, rowsum_sh)

    plsc.subcore_barrier()

    @pl.when(s != 0)
    def _():
        pltpu.sync_copy(rs_loc, rowsum_sh.at[idx2.at[0]], add=True)

    plsc.subcore_barrier()
    pltpu.sync_copy(rowsum_sh, rs_loc)

    @pl.loop(0, C)
    def _(i):
        for j in range(H // L):
            rows_buf[i, pl.ds(j * L, L)] = zeros16

    for k in range(RPT // C):
        pltpu.sync_copy(rows_buf, out_sh.at[pl.ds(s * RPT + k * C, C), :])

    plsc.subcore_barrier()

    @pl.loop(0, NCB)
    def _(cb):
        pltpu.sync_copy(row4.at[w_own, cb], row_cb)
        pltpu.sync_copy(col4.at[w_own, cb], col_cb)
        pltpu.sync_copy(adj4.at[w_own, cb], adj_cb)

        @pl.loop(0, KC)
        def _(k):
            for j in range(C // L):
                r = row_cb[k, pl.ds(j * L, L)]
                cc = col_cb[k, pl.ds(j * L, L)]
                e = jnp.exp(plsc.load_gather(a1b, [r]) +
                            plsc.load_gather(a2b, [cc]))
                rs = plsc.load_gather(rs_loc, _split(r))
                av = adj_cb[k, pl.ds(j * L, L)]
                vals_cb[k, pl.ds(j * L, L)] = (
                    (e / (rs + 1e-6) + 0.5 * av) * (1.0 / 1.5))

            pltpu.sync_copy(emb_hbm.at[col_cb.at[k]], rows_buf)
            for jb in range(C // L):
                v16 = vals_cb[k, pl.ds(jb * L, L)]
                for i in range(L):
                    ri = jb * L + i
                    v = v16[i]
                    for j2 in range(H // L):
                        sl = pl.ds(j2 * L, L)
                        rows_buf[ri, sl] = rows_buf[ri, sl] * v
            pltpu.sync_copy(rows_buf, out_sh.at[row_cb.at[k]], add=True)

        pltpu.sync_copy(vals_cb, values_hbm.at[w_own, cb])

    plsc.subcore_barrier()
    pltpu.sync_copy(out_sh.at[pl.ds(s * RPT, RPT), :],
                    partial_hbm.at[c, pl.ds(s * RPT, RPT), :])


@functools.cache
def _make_sc_call():
    return pl.kernel(
        _sc_body,
        out_type=(
            jax.ShapeDtypeStruct((NW, NCB, KC, C), jnp.float32),
            jax.ShapeDtypeStruct((NC, NP2, H), jnp.float32),
        ),
        mesh=plsc.VectorSubcoreMesh(core_axis_name="c", subcore_axis_name="s",
                                    num_cores=NC, num_subcores=NS),
        compiler_params=pltpu.CompilerParams(needs_layout_passes=False),
        scratch_types=(
            pltpu.VMEM((N,), jnp.float32),
            pltpu.VMEM((N,), jnp.float32),
            pltpu.VMEM((RS_R, 128), jnp.float32),
            pltpu.VMEM((KC, C), jnp.int32),
            pltpu.VMEM((KC, C), jnp.int32),
            pltpu.VMEM((KC, C), jnp.float32),
            pltpu.VMEM((KC, C), jnp.float32),
            pltpu.VMEM((C, H), jnp.float32),
            pltpu.VMEM((1, RS_R), jnp.int32),
            pltpu.VMEM_SHARED((RS_R, 128), jnp.float32),
            pltpu.VMEM_SHARED((NP2, H), jnp.float32),
        ),
    )


def _sum_body(p_ref, o_ref):
    o_ref[:, :] = p_ref[0] + p_ref[1]


def _sum_partials(partial):
    blk = 2000
    return pl.pallas_call(
        _sum_body,
        grid=(N // blk,),
        in_specs=[pl.BlockSpec((NC, blk, H), lambda i: (0, i, 0))],
        out_specs=pl.BlockSpec((blk, H), lambda i: (i, 0)),
        out_shape=jax.ShapeDtypeStruct((N, H), jnp.float32),
    )(partial)


@jax.jit
def kernel(edge_index, adj_values, embeds, W1, b1, W2, b2, Wa, ba):
    row = edge_index[0, :].astype(jnp.int32)
    col = edge_index[1, :].astype(jnp.int32)
    wa1 = Wa[:H, 0].reshape(1, H)
    wa2 = Wa[H:, 0].reshape(1, H)

    a1, a2 = _node_scores(embeds, W1, b1, W2, b2, wa1, wa2, ba)

    row4 = row.reshape(NW, NCB, KC, C)
    col4 = col.reshape(NW, NCB, KC, C)
    adj4 = adj_values.reshape(NW, NCB, KC, C)
    values, partial = _make_sc_call()(row4, col4, adj4, a1, a2, embeds)
    out = _sum_partials(partial)
    return values.reshape(E), out


# speedup vs baseline: 18.9929x; 1.6480x over previous
import functools

import jax
import jax.numpy as jnp
from jax import lax
from jax.experimental import pallas as pl
from jax.experimental.pallas import tpu as pltpu
from jax.experimental.pallas import tpu_sc as plsc

N = 10000
E = 320000
H = 128
NC = 2
NS = 16
L = 16
NW = NC * NS
EW = E // NW
C = 80
NP2 = 10240
RPT = NP2 // NS


def _scores_body(emb_ref, w1_ref, b1_ref, w2_ref, b2_ref, wa1_ref, wa2_ref,
                 ba_ref, a1_ref, a2_ref):
    emb = emb_ref[:, :]
    h1 = jnp.maximum(
        jnp.dot(emb, w1_ref[:, :], preferred_element_type=jnp.float32,
                precision=lax.Precision.HIGHEST) + b1_ref[:, :], 0.0)
    h2 = jnp.maximum(
        jnp.dot(emb, w2_ref[:, :], preferred_element_type=jnp.float32,
                precision=lax.Precision.HIGHEST) + b2_ref[:, :], 0.0)
    a1_ref[:, :] = jnp.sum(h1 * wa1_ref[:, :], axis=1, keepdims=True) + ba_ref[0, 0]
    a2_ref[:, :] = jnp.sum(h2 * wa2_ref[:, :], axis=1, keepdims=True)


def _node_scores(embeds, W1, b1, W2, b2, wa1, wa2, ba):
    blk = 2000
    grid = N // blk
    a1, a2 = pl.pallas_call(
        _scores_body,
        grid=(grid,),
        in_specs=[
            pl.BlockSpec((blk, H), lambda i: (i, 0)),
            pl.BlockSpec((H, H), lambda i: (0, 0)),
            pl.BlockSpec((1, H), lambda i: (0, 0)),
            pl.BlockSpec((H, H), lambda i: (0, 0)),
            pl.BlockSpec((1, H), lambda i: (0, 0)),
            pl.BlockSpec((1, H), lambda i: (0, 0)),
            pl.BlockSpec((1, H), lambda i: (0, 0)),
            pl.BlockSpec(memory_space=pltpu.SMEM),
        ],
        out_specs=[
            pl.BlockSpec((blk, 1), lambda i: (i, 0)),
            pl.BlockSpec((blk, 1), lambda i: (i, 0)),
        ],
        out_shape=[
            jax.ShapeDtypeStruct((N, 1), jnp.float32),
            jax.ShapeDtypeStruct((N, 1), jnp.float32),
        ],
    )(embeds, W1, b1.reshape(1, H), W2, b2.reshape(1, H), wa1, wa2,
      ba.reshape(1, 1))
    return a1.reshape(N), a2.reshape(N)


KC = 5
CB = KC * C
NCB = EW // CB
RS_R = 80
NBUF = 3


def _split(r):
    return [lax.shift_right_logical(r, 7), lax.bitwise_and(r, 127)]


def _sca_body(row4, col4, adj4, a1_hbm, a2_hbm,
              vals4,
              a1b, a2b, rs_loc, row_b, col_b, adj_b,
              vals_b, idx2,
              rcsem, asem, wsem,
              rowsum_sh):
    s = lax.axis_index("s")
    c = lax.axis_index("c")
    w_own = c * NS + s
    w_mir = (1 - c) * NS + s

    pltpu.sync_copy(a1_hbm, a1b)
    pltpu.sync_copy(a2_hbm, a2b)

    zeros16 = jnp.zeros((L,), jnp.float32)

    @pl.loop(0, RS_R)
    def _(i):
        for j in range(128 // L):
            rs_loc[i, pl.ds(j * L, L)] = zeros16

    for k in range(RS_R // L):
        idx2[0, pl.ds(k * L, L)] = lax.iota(jnp.int32, L) + k * L

    def _load_rc(w, cb, p, sem_i):
        band = pl.ds(p * KC, KC)
        pltpu.async_copy(row4.at[w, cb], row_b.at[band, :], rcsem.at[sem_i])
        pltpu.async_copy(col4.at[w, cb], col_b.at[band, :], rcsem.at[sem_i])

    def _wait_rc(p, sem_i):
        band = pl.ds(p * KC, KC)
        pltpu.make_async_copy(row4.at[0, 0], row_b.at[band, :],
                              rcsem.at[sem_i]).wait()
        pltpu.make_async_copy(col4.at[0, 0], col_b.at[band, :],
                              rcsem.at[sem_i]).wait()

    _load_rc(w_own, 0, 0, 0)

    @pl.loop(0, 2 * NCB)
    def _(t):
        p = lax.bitwise_and(t, 1)
        _wait_rc(p, p)

        @pl.when(t + 1 < 2 * NCB)
        def _():
            tn = t + 1
            wn = jnp.where(tn < NCB, w_own, w_mir)
            cbn = jnp.where(tn < NCB, tn, tn - NCB)
            _load_rc(wn, cbn, 1 - p, 1 - p)

        @pl.loop(0, KC)
        def _(k):
            kk = p * KC + k
            for j in range(C // L):
                r = row_b[kk, pl.ds(j * L, L)]
                cc = col_b[kk, pl.ds(j * L, L)]
                e = jnp.exp(plsc.load_gather(a1b, [r]) +
                            plsc.load_gather(a2b, [cc]))
                plsc.addupdate_scatter(rs_loc, _split(r), e)

    @pl.when(s == 0)
    def _():
        pltpu.sync_copy(rs_loc, rowsum_sh)

    plsc.subcore_barrier()

    @pl.when(s != 0)
    def _():
        pltpu.sync_copy(rs_loc, rowsum_sh.at[idx2.at[0]], add=True)

    plsc.subcore_barrier()
    pltpu.sync_copy(rowsum_sh, rs_loc)

    _load_rc(w_own, 0, 0, 0)
    pltpu.async_copy(adj4.at[w_own, 0], adj_b.at[pl.ds(0, KC), :], asem.at[0])

    @pl.loop(0, NCB)
    def _(cb):
        p = lax.bitwise_and(cb, 1)
        band = pl.ds(p * KC, KC)
        _wait_rc(p, p)
        pltpu.make_async_copy(adj4.at[0, 0], adj_b.at[band, :],
                              asem.at[p]).wait()

        @pl.when(cb + 1 < NCB)
        def _():
            nband = pl.ds((1 - p) * KC, KC)
            _load_rc(w_own, cb + 1, 1 - p, 1 - p)
            pltpu.async_copy(adj4.at[w_own, cb + 1], adj_b.at[nband, :],
                             asem.at[1 - p])

        @pl.when(cb >= 2)
        def _():
            pltpu.make_async_copy(vals_b.at[band, :], vals4.at[w_own, 0],
                                  wsem.at[p]).wait()

        @pl.loop(0, KC)
        def _(k):
            kk = p * KC + k
            for j in range(C // L):
                r = row_b[kk, pl.ds(j * L, L)]
                cc = col_b[kk, pl.ds(j * L, L)]
                e = jnp.exp(plsc.load_gather(a1b, [r]) +
                            plsc.load_gather(a2b, [cc]))
                rs = plsc.load_gather(rs_loc, _split(r))
                av = adj_b[kk, pl.ds(j * L, L)]
                vals_b[kk, pl.ds(j * L, L)] = (
                    (e / (rs + 1e-6) + 0.5 * av) * (1.0 / 1.5))

        pltpu.async_copy(vals_b.at[band, :], vals4.at[w_own, cb], wsem.at[p])

    for p in range(2):
        pltpu.make_async_copy(vals_b.at[pl.ds(p * KC, KC), :],
                              vals4.at[w_own, 0], wsem.at[p]).wait()


def _scb_body(row4, col4, vals4, emb_hbm,
              partial_hbm,
              row_b, col_b, vals_b, rows3,
              rcsem, vsem, gsem, ssem,
              out_sh):
    s = lax.axis_index("s")
    c = lax.axis_index("c")
    w_own = c * NS + s

    zeros16 = jnp.zeros((L,), jnp.float32)

    @pl.loop(0, C)
    def _(i):
        for j in range(H // L):
            rows3[0, i, pl.ds(j * L, L)] = zeros16

    for k in range(RPT // C):
        pltpu.sync_copy(rows3.at[0], out_sh.at[pl.ds(s * RPT + k * C, C), :])

    plsc.subcore_barrier()

    def _load_rcv(cb, p):
        band = pl.ds(p * KC, KC)
        pltpu.async_copy(row4.at[w_own, cb], row_b.at[band, :], rcsem.at[p])
        pltpu.async_copy(col4.at[w_own, cb], col_b.at[band, :], rcsem.at[p])
        pltpu.async_copy(vals4.at[w_own, cb], vals_b.at[band, :], vsem.at[p])

    def _wait_rcv(p):
        band = pl.ds(p * KC, KC)
        pltpu.make_async_copy(row4.at[0, 0], row_b.at[band, :],
                              rcsem.at[p]).wait()
        pltpu.make_async_copy(col4.at[0, 0], col_b.at[band, :],
                              rcsem.at[p]).wait()
        pltpu.make_async_copy(vals4.at[0, 0], vals_b.at[band, :],
                              vsem.at[p]).wait()

    _load_rcv(0, 0)

    @pl.loop(0, NCB)
    def _(cb):
        p = lax.bitwise_and(cb, 1)
        _wait_rcv(p)

        @pl.when(cb + 1 < NCB)
        def _():
            _load_rcv(cb + 1, 1 - p)

        def _gather(k, q):
            pltpu.async_copy(emb_hbm.at[col_b.at[p * KC + k]], rows3.at[q],
                             gsem.at[q])

        _gather(0, 0)
        _gather(1, 1)

        @pl.loop(0, KC)
        def _(k):
            q = lax.rem(k, NBUF)
            kk = p * KC + k
            pltpu.make_async_copy(emb_hbm.at[col_b.at[0]],
                                  rows3.at[q], gsem.at[q]).wait()

            for jb in range(C // L):
                v16 = vals_b[kk, pl.ds(jb * L, L)]
                for i in range(L):
                    ri = jb * L + i
                    v = v16[i]
                    for j2 in range(H // L):
                        sl = pl.ds(j2 * L, L)
                        rows3[q, ri, sl] = rows3[q, ri, sl] * v

            pltpu.async_copy(rows3.at[q], out_sh.at[row_b.at[kk]],
                             ssem.at[q], add=True)

            @pl.when(jnp.logical_and(k >= 1, k + 2 < KC))
            def _():
                qn = lax.rem(k + 2, NBUF)
                pltpu.make_async_copy(rows3.at[qn], out_sh.at[row_b.at[0]],
                                      ssem.at[qn]).wait()

            @pl.when(k + 2 < KC)
            def _():
                _gather(k + 2, lax.rem(k + 2, NBUF))

        for q in range(NBUF):
            pltpu.make_async_copy(rows3.at[q], out_sh.at[row_b.at[0]],
                                  ssem.at[q]).wait()

    plsc.subcore_barrier()
    pltpu.sync_copy(out_sh.at[pl.ds(s * RPT, RPT), :],
                    partial_hbm.at[c, pl.ds(s * RPT, RPT), :])


@functools.cache
def _make_sca_call():
    return pl.kernel(
        _sca_body,
        out_type=jax.ShapeDtypeStruct((NW, NCB, KC, C), jnp.float32),
        mesh=plsc.VectorSubcoreMesh(core_axis_name="c", subcore_axis_name="s",
                                    num_cores=NC, num_subcores=NS),
        compiler_params=pltpu.CompilerParams(needs_layout_passes=False),
        scratch_types=(
            pltpu.VMEM((N,), jnp.float32),
            pltpu.VMEM((N,), jnp.float32),
            pltpu.VMEM((RS_R, 128), jnp.float32),
            pltpu.VMEM((2 * KC, C), jnp.int32),
            pltpu.VMEM((2 * KC, C), jnp.int32),
            pltpu.VMEM((2 * KC, C), jnp.float32),
            pltpu.VMEM((2 * KC, C), jnp.float32),
            pltpu.VMEM((1, RS_R), jnp.int32),
            pltpu.SemaphoreType.DMA((2,)),
            pltpu.SemaphoreType.DMA((2,)),
            pltpu.SemaphoreType.DMA((2,)),
            pltpu.VMEM_SHARED((RS_R, 128), jnp.float32),
        ),
    )


@functools.cache
def _make_scb_call():
    return pl.kernel(
        _scb_body,
        out_type=jax.ShapeDtypeStruct((NC, NP2, H), jnp.float32),
        mesh=plsc.VectorSubcoreMesh(core_axis_name="c", subcore_axis_name="s",
                                    num_cores=NC, num_subcores=NS),
        compiler_params=pltpu.CompilerParams(needs_layout_passes=False),
        scratch_types=(
            pltpu.VMEM((2 * KC, C), jnp.int32),
            pltpu.VMEM((2 * KC, C), jnp.int32),
            pltpu.VMEM((2 * KC, C), jnp.float32),
            pltpu.VMEM((NBUF, C, H), jnp.float32),
            pltpu.SemaphoreType.DMA((2,)),
            pltpu.SemaphoreType.DMA((2,)),
            pltpu.SemaphoreType.DMA((NBUF,)),
            pltpu.SemaphoreType.DMA((NBUF,)),
            pltpu.VMEM_SHARED((NP2, H), jnp.float32),
        ),
    )


def _sum_body(p_ref, o_ref):
    o_ref[:, :] = p_ref[0] + p_ref[1]


def _sum_partials(partial):
    blk = 2000
    return pl.pallas_call(
        _sum_body,
        grid=(N // blk,),
        in_specs=[pl.BlockSpec((NC, blk, H), lambda i: (0, i, 0))],
        out_specs=pl.BlockSpec((blk, H), lambda i: (i, 0)),
        out_shape=jax.ShapeDtypeStruct((N, H), jnp.float32),
    )(partial)


@jax.jit
def kernel(edge_index, adj_values, embeds, W1, b1, W2, b2, Wa, ba):
    row = edge_index[0, :].astype(jnp.int32)
    col = edge_index[1, :].astype(jnp.int32)
    wa1 = Wa[:H, 0].reshape(1, H)
    wa2 = Wa[H:, 0].reshape(1, H)

    a1, a2 = _node_scores(embeds, W1, b1, W2, b2, wa1, wa2, ba)

    row4 = row.reshape(NW, NCB, KC, C)
    col4 = col.reshape(NW, NCB, KC, C)
    adj4 = adj_values.reshape(NW, NCB, KC, C)

    values = _make_sca_call()(row4, col4, adj4, a1, a2)
    partial = _make_scb_call()(row4, col4, values, embeds)
    out = _sum_partials(partial)
    return values.reshape(E), out


# speedup vs baseline: 21.1246x; 1.1122x over previous
import functools

import jax
import jax.numpy as jnp
from jax import lax
from jax.experimental import pallas as pl
from jax.experimental.pallas import tpu as pltpu
from jax.experimental.pallas import tpu_sc as plsc

N = 10000
E = 320000
H = 128
NC = 2
NS = 16
L = 16
NW = NC * NS
EW = E // NW
C = 80
NP2 = 10240
RPT = NP2 // NS


def _scores_body(emb_ref, w1_ref, b1_ref, w2_ref, b2_ref, wa1_ref, wa2_ref,
                 ba_ref, a1_ref, a2_ref):
    emb = emb_ref[:, :]
    h1 = jnp.maximum(
        jnp.dot(emb, w1_ref[:, :], preferred_element_type=jnp.float32,
                precision=lax.Precision.HIGHEST) + b1_ref[:, :], 0.0)
    h2 = jnp.maximum(
        jnp.dot(emb, w2_ref[:, :], preferred_element_type=jnp.float32,
                precision=lax.Precision.HIGHEST) + b2_ref[:, :], 0.0)
    a1_ref[:, :] = jnp.sum(h1 * wa1_ref[:, :], axis=1, keepdims=True) + ba_ref[0, 0]
    a2_ref[:, :] = jnp.sum(h2 * wa2_ref[:, :], axis=1, keepdims=True)


def _node_scores(embeds, W1, b1, W2, b2, wa1, wa2, ba):
    blk = 2000
    grid = N // blk
    a1, a2 = pl.pallas_call(
        _scores_body,
        grid=(grid,),
        in_specs=[
            pl.BlockSpec((blk, H), lambda i: (i, 0)),
            pl.BlockSpec((H, H), lambda i: (0, 0)),
            pl.BlockSpec((1, H), lambda i: (0, 0)),
            pl.BlockSpec((H, H), lambda i: (0, 0)),
            pl.BlockSpec((1, H), lambda i: (0, 0)),
            pl.BlockSpec((1, H), lambda i: (0, 0)),
            pl.BlockSpec((1, H), lambda i: (0, 0)),
            pl.BlockSpec(memory_space=pltpu.SMEM),
        ],
        out_specs=[
            pl.BlockSpec((blk, 1), lambda i: (i, 0)),
            pl.BlockSpec((blk, 1), lambda i: (i, 0)),
        ],
        out_shape=[
            jax.ShapeDtypeStruct((N, 1), jnp.float32),
            jax.ShapeDtypeStruct((N, 1), jnp.float32),
        ],
    )(embeds, W1, b1.reshape(1, H), W2, b2.reshape(1, H), wa1, wa2,
      ba.reshape(1, 1))
    return a1.reshape(N), a2.reshape(N)


KC = 5
CB = KC * C
NCB = EW // CB
RS_R = 80
NBUF = 4


def _split(r):
    return [lax.shift_right_logical(r, 7), lax.bitwise_and(r, 127)]


def _sca_body(row4, col4, adj4, a1_hbm, a2_hbm,
              vals4,
              a1b, a2b, rs_loc, row_b, col_b, adj_b,
              vals_b, idx2,
              rcsem, asem, wsem,
              rowsum_sh):
    s = lax.axis_index("s")
    c = lax.axis_index("c")
    w_own = c * NS + s
    w_mir = (1 - c) * NS + s

    pltpu.sync_copy(a1_hbm, a1b)
    pltpu.sync_copy(a2_hbm, a2b)

    zeros16 = jnp.zeros((L,), jnp.float32)

    @pl.loop(0, RS_R)
    def _(i):
        for j in range(128 // L):
            rs_loc[i, pl.ds(j * L, L)] = zeros16

    for k in range(RS_R // L):
        idx2[0, pl.ds(k * L, L)] = lax.iota(jnp.int32, L) + k * L

    def _load_rc(w, cb, p, sem_i):
        band = pl.ds(p * KC, KC)
        pltpu.async_copy(row4.at[w, cb], row_b.at[band, :], rcsem.at[sem_i])
        pltpu.async_copy(col4.at[w, cb], col_b.at[band, :], rcsem.at[sem_i])

    def _wait_rc(p, sem_i):
        band = pl.ds(p * KC, KC)
        pltpu.make_async_copy(row4.at[0, 0], row_b.at[band, :],
                              rcsem.at[sem_i]).wait()
        pltpu.make_async_copy(col4.at[0, 0], col_b.at[band, :],
                              rcsem.at[sem_i]).wait()

    _load_rc(w_own, 0, 0, 0)

    @pl.loop(0, 2 * NCB)
    def _(t):
        p = lax.bitwise_and(t, 1)
        _wait_rc(p, p)

        @pl.when(t + 1 < 2 * NCB)
        def _():
            tn = t + 1
            wn = jnp.where(tn < NCB, w_own, w_mir)
            cbn = jnp.where(tn < NCB, tn, tn - NCB)
            _load_rc(wn, cbn, 1 - p, 1 - p)

        @pl.loop(0, KC)
        def _(k):
            kk = p * KC + k
            for j in range(C // L):
                r = row_b[kk, pl.ds(j * L, L)]
                cc = col_b[kk, pl.ds(j * L, L)]
                e = jnp.exp(plsc.load_gather(a1b, [r]) +
                            plsc.load_gather(a2b, [cc]))
                plsc.addupdate_scatter(rs_loc, _split(r), e)

    @pl.when(s == 0)
    def _():
        pltpu.sync_copy(rs_loc, rowsum_sh)

    plsc.subcore_barrier()

    @pl.when(s != 0)
    def _():
        pltpu.sync_copy(rs_loc, rowsum_sh.at[idx2.at[0]], add=True)

    plsc.subcore_barrier()
    pltpu.sync_copy(rowsum_sh, rs_loc)

    _load_rc(w_own, 0, 0, 0)
    pltpu.async_copy(adj4.at[w_own, 0], adj_b.at[pl.ds(0, KC), :], asem.at[0])

    @pl.loop(0, NCB)
    def _(cb):
        p = lax.bitwise_and(cb, 1)
        band = pl.ds(p * KC, KC)
        _wait_rc(p, p)
        pltpu.make_async_copy(adj4.at[0, 0], adj_b.at[band, :],
                              asem.at[p]).wait()

        @pl.when(cb + 1 < NCB)
        def _():
            nband = pl.ds((1 - p) * KC, KC)
            _load_rc(w_own, cb + 1, 1 - p, 1 - p)
            pltpu.async_copy(adj4.at[w_own, cb + 1], adj_b.at[nband, :],
                             asem.at[1 - p])

        @pl.when(cb >= 2)
        def _():
            pltpu.make_async_copy(vals_b.at[band, :], vals4.at[w_own, 0],
                                  wsem.at[p]).wait()

        @pl.loop(0, KC)
        def _(k):
            kk = p * KC + k
            for j in range(C // L):
                r = row_b[kk, pl.ds(j * L, L)]
                cc = col_b[kk, pl.ds(j * L, L)]
                e = jnp.exp(plsc.load_gather(a1b, [r]) +
                            plsc.load_gather(a2b, [cc]))
                rs = plsc.load_gather(rs_loc, _split(r))
                av = adj_b[kk, pl.ds(j * L, L)]
                vals_b[kk, pl.ds(j * L, L)] = (
                    (e / (rs + 1e-6) + 0.5 * av) * (1.0 / 1.5))

        pltpu.async_copy(vals_b.at[band, :], vals4.at[w_own, cb], wsem.at[p])

    for p in range(2):
        pltpu.make_async_copy(vals_b.at[pl.ds(p * KC, KC), :],
                              vals4.at[w_own, 0], wsem.at[p]).wait()


def _scb_body(row4, col4, vals4, emb_hbm,
              partial_hbm,
              row_b, col_b, vals_b, rows3,
              rcsem, vsem, gsem, ssem,
              out_sh):
    s = lax.axis_index("s")
    c = lax.axis_index("c")
    w_own = c * NS + s

    zeros16 = jnp.zeros((L,), jnp.float32)

    @pl.loop(0, C)
    def _(i):
        for j in range(H // L):
            rows3[0, i, pl.ds(j * L, L)] = zeros16

    for k in range(RPT // C):
        pltpu.sync_copy(rows3.at[0], out_sh.at[pl.ds(s * RPT + k * C, C), :])

    plsc.subcore_barrier()

    def _load_rcv(cb, p):
        band = pl.ds(p * KC, KC)
        pltpu.async_copy(row4.at[w_own, cb], row_b.at[band, :], rcsem.at[p])
        pltpu.async_copy(col4.at[w_own, cb], col_b.at[band, :], rcsem.at[p])
        pltpu.async_copy(vals4.at[w_own, cb], vals_b.at[band, :], vsem.at[p])

    def _wait_rcv(p):
        band = pl.ds(p * KC, KC)
        pltpu.make_async_copy(row4.at[0, 0], row_b.at[band, :],
                              rcsem.at[p]).wait()
        pltpu.make_async_copy(col4.at[0, 0], col_b.at[band, :],
                              rcsem.at[p]).wait()
        pltpu.make_async_copy(vals4.at[0, 0], vals_b.at[band, :],
                              vsem.at[p]).wait()

    _load_rcv(0, 0)
    _wait_rcv(0)

    def _gather(t, q):
        cbp = lax.bitwise_and(lax.div(t, KC), 1)
        pltpu.async_copy(emb_hbm.at[col_b.at[cbp * KC + lax.rem(t, KC)]],
                         rows3.at[q], gsem.at[q])

    _gather(jnp.int32(0), jnp.int32(0))
    _gather(jnp.int32(1), jnp.int32(1))

    NT = EW // C

    @pl.loop(0, NT)
    def _(t):
        k = lax.rem(t, KC)
        cb = lax.div(t, KC)
        p = lax.bitwise_and(cb, 1)
        q = lax.rem(t, NBUF)
        kk = p * KC + k

        pltpu.make_async_copy(emb_hbm.at[col_b.at[0]],
                              rows3.at[q], gsem.at[q]).wait()

        for jb in range(C // L):
            v16 = vals_b[kk, pl.ds(jb * L, L)]
            for i in range(L):
                ri = jb * L + i
                v = v16[i]
                for j2 in range(H // L):
                    sl = pl.ds(j2 * L, L)
                    rows3[q, ri, sl] = rows3[q, ri, sl] * v

        pltpu.async_copy(rows3.at[q], out_sh.at[row_b.at[kk]],
                         ssem.at[q], add=True)

        @pl.when(t + 2 < NT)
        def _():
            qn = lax.rem(t + 2, NBUF)

            @pl.when(t >= 2)
            def _():
                pltpu.make_async_copy(rows3.at[qn], out_sh.at[row_b.at[0]],
                                      ssem.at[qn]).wait()

            @pl.when(lax.rem(t + 2, KC) == 0)
            def _():
                _wait_rcv(1 - p)

            _gather(t + 2, qn)

        @pl.when(jnp.logical_and(k == 1, cb + 1 < NCB))
        def _():
            _load_rcv(cb + 1, 1 - p)

    for q in range(NBUF):
        pltpu.make_async_copy(rows3.at[q], out_sh.at[row_b.at[0]],
                              ssem.at[q]).wait()

    plsc.subcore_barrier()
    pltpu.sync_copy(out_sh.at[pl.ds(s * RPT, RPT), :],
                    partial_hbm.at[c, pl.ds(s * RPT, RPT), :])


@functools.cache
def _make_sca_call():
    return pl.kernel(
        _sca_body,
        out_type=jax.ShapeDtypeStruct((NW, NCB, KC, C), jnp.float32),
        mesh=plsc.VectorSubcoreMesh(core_axis_name="c", subcore_axis_name="s",
                                    num_cores=NC, num_subcores=NS),
        compiler_params=pltpu.CompilerParams(needs_layout_passes=False),
        scratch_types=(
            pltpu.VMEM((N,), jnp.float32),
            pltpu.VMEM((N,), jnp.float32),
            pltpu.VMEM((RS_R, 128), jnp.float32),
            pltpu.VMEM((2 * KC, C), jnp.int32),
            pltpu.VMEM((2 * KC, C), jnp.int32),
            pltpu.VMEM((2 * KC, C), jnp.float32),
            pltpu.VMEM((2 * KC, C), jnp.float32),
            pltpu.VMEM((1, RS_R), jnp.int32),
            pltpu.SemaphoreType.DMA((2,)),
            pltpu.SemaphoreType.DMA((2,)),
            pltpu.SemaphoreType.DMA((2,)),
            pltpu.VMEM_SHARED((RS_R, 128), jnp.float32),
        ),
    )


@functools.cache
def _make_scb_call():
    return pl.kernel(
        _scb_body,
        out_type=jax.ShapeDtypeStruct((NC, NP2, H), jnp.float32),
        mesh=plsc.VectorSubcoreMesh(core_axis_name="c", subcore_axis_name="s",
                                    num_cores=NC, num_subcores=NS),
        compiler_params=pltpu.CompilerParams(needs_layout_passes=False),
        scratch_types=(
            pltpu.VMEM((2 * KC, C), jnp.int32),
            pltpu.VMEM((2 * KC, C), jnp.int32),
            pltpu.VMEM((2 * KC, C), jnp.float32),
            pltpu.VMEM((NBUF, C, H), jnp.float32),
            pltpu.SemaphoreType.DMA((2,)),
            pltpu.SemaphoreType.DMA((2,)),
            pltpu.SemaphoreType.DMA((NBUF,)),
            pltpu.SemaphoreType.DMA((NBUF,)),
            pltpu.VMEM_SHARED((NP2, H), jnp.float32),
        ),
    )


def _sum_body(p_ref, o_ref):
    o_ref[:, :] = p_ref[0] + p_ref[1]


def _sum_partials(partial):
    blk = 2000
    return pl.pallas_call(
        _sum_body,
        grid=(N // blk,),
        in_specs=[pl.BlockSpec((NC, blk, H), lambda i: (0, i, 0))],
        out_specs=pl.BlockSpec((blk, H), lambda i: (i, 0)),
        out_shape=jax.ShapeDtypeStruct((N, H), jnp.float32),
    )(partial)


@jax.jit
def kernel(edge_index, adj_values, embeds, W1, b1, W2, b2, Wa, ba):
    row = edge_index[0, :].astype(jnp.int32)
    col = edge_index[1, :].astype(jnp.int32)
    wa1 = Wa[:H, 0].reshape(1, H)
    wa2 = Wa[H:, 0].reshape(1, H)

    a1, a2 = _node_scores(embeds, W1, b1, W2, b2, wa1, wa2, ba)

    row4 = row.reshape(NW, NCB, KC, C)
    col4 = col.reshape(NW, NCB, KC, C)
    adj4 = adj_values.reshape(NW, NCB, KC, C)

    values = _make_sca_call()(row4, col4, adj4, a1, a2)
    partial = _make_scb_call()(row4, col4, values, embeds)
    out = _sum_partials(partial)
    return values.reshape(E), out


# speedup vs baseline: 21.5745x; 1.0213x over previous
import functools

import jax
import jax.numpy as jnp
from jax import lax
from jax.experimental import pallas as pl
from jax.experimental.pallas import tpu as pltpu
from jax.experimental.pallas import tpu_sc as plsc

N = 10000
E = 320000
H = 128
NC = 2
NS = 16
L = 16
NW = NC * NS
EW = E // NW
C = 80
NP2 = 10240
RPT = NP2 // NS


def _scores_body(emb_ref, w1_ref, b1_ref, w2_ref, b2_ref, wa1_ref, wa2_ref,
                 ba_ref, a1_ref, a2_ref):
    emb = emb_ref[:, :]
    h1 = jnp.maximum(
        jnp.dot(emb, w1_ref[:, :], preferred_element_type=jnp.float32,
                precision=lax.Precision.HIGHEST) + b1_ref[:, :], 0.0)
    h2 = jnp.maximum(
        jnp.dot(emb, w2_ref[:, :], preferred_element_type=jnp.float32,
                precision=lax.Precision.HIGHEST) + b2_ref[:, :], 0.0)
    a1_ref[:, :] = jnp.sum(h1 * wa1_ref[:, :], axis=1, keepdims=True) + ba_ref[0, 0]
    a2_ref[:, :] = jnp.sum(h2 * wa2_ref[:, :], axis=1, keepdims=True)


def _node_scores(embeds, W1, b1, W2, b2, wa1, wa2, ba):
    blk = 2000
    grid = N // blk
    a1, a2 = pl.pallas_call(
        _scores_body,
        grid=(grid,),
        in_specs=[
            pl.BlockSpec((blk, H), lambda i: (i, 0)),
            pl.BlockSpec((H, H), lambda i: (0, 0)),
            pl.BlockSpec((1, H), lambda i: (0, 0)),
            pl.BlockSpec((H, H), lambda i: (0, 0)),
            pl.BlockSpec((1, H), lambda i: (0, 0)),
            pl.BlockSpec((1, H), lambda i: (0, 0)),
            pl.BlockSpec((1, H), lambda i: (0, 0)),
            pl.BlockSpec(memory_space=pltpu.SMEM),
        ],
        out_specs=[
            pl.BlockSpec((blk, 1), lambda i: (i, 0)),
            pl.BlockSpec((blk, 1), lambda i: (i, 0)),
        ],
        out_shape=[
            jax.ShapeDtypeStruct((N, 1), jnp.float32),
            jax.ShapeDtypeStruct((N, 1), jnp.float32),
        ],
    )(embeds, W1, b1.reshape(1, H), W2, b2.reshape(1, H), wa1, wa2,
      ba.reshape(1, 1))
    return a1.reshape(N), a2.reshape(N)


KC = 5
CB = KC * C
NCB = EW // CB
KCA = 25
NCBA = EW // (KCA * C)
RS_R = 80
NBUF = 4


def _split(r):
    return [lax.shift_right_logical(r, 7), lax.bitwise_and(r, 127)]


def _sca_body(row4, col4, adj4, a1_hbm, a2_hbm,
              vals4,
              a1b, a2b, rs_loc, row_b, col_b, adj_b,
              vals_b, idx2,
              rcsem, asem, wsem,
              rowsum_sh):
    s = lax.axis_index("s")
    c = lax.axis_index("c")
    w_own = c * NS + s
    w_mir = (1 - c) * NS + s

    pltpu.sync_copy(a1_hbm, a1b)
    pltpu.sync_copy(a2_hbm, a2b)

    zeros16 = jnp.zeros((L,), jnp.float32)

    @pl.loop(0, RS_R)
    def _(i):
        for j in range(128 // L):
            rs_loc[i, pl.ds(j * L, L)] = zeros16

    for k in range(RS_R // L):
        idx2[0, pl.ds(k * L, L)] = lax.iota(jnp.int32, L) + k * L

    def _load_rc(w, cb, p, sem_i):
        band = pl.ds(p * KCA, KCA)
        pltpu.async_copy(row4.at[w, cb], row_b.at[band, :], rcsem.at[sem_i])
        pltpu.async_copy(col4.at[w, cb], col_b.at[band, :], rcsem.at[sem_i])

    def _wait_rc(p, sem_i):
        band = pl.ds(p * KCA, KCA)
        pltpu.make_async_copy(row4.at[0, 0], row_b.at[band, :],
                              rcsem.at[sem_i]).wait()
        pltpu.make_async_copy(col4.at[0, 0], col_b.at[band, :],
                              rcsem.at[sem_i]).wait()

    _load_rc(w_own, 0, 0, 0)

    @pl.loop(0, 2 * NCBA)
    def _(t):
        p = lax.bitwise_and(t, 1)
        _wait_rc(p, p)

        @pl.when(t + 1 < 2 * NCBA)
        def _():
            tn = t + 1
            wn = jnp.where(tn < NCBA, w_own, w_mir)
            cbn = jnp.where(tn < NCBA, tn, tn - NCBA)
            _load_rc(wn, cbn, 1 - p, 1 - p)

        @pl.loop(0, KCA)
        def _(k):
            kk = p * KCA + k
            for j in range(C // L):
                r = row_b[kk, pl.ds(j * L, L)]
                cc = col_b[kk, pl.ds(j * L, L)]
                e = jnp.exp(plsc.load_gather(a1b, [r]) +
                            plsc.load_gather(a2b, [cc]))
                plsc.addupdate_scatter(rs_loc, _split(r), e)

    @pl.when(s == 0)
    def _():
        pltpu.sync_copy(rs_loc, rowsum_sh)

    plsc.subcore_barrier()

    @pl.when(s != 0)
    def _():
        pltpu.sync_copy(rs_loc, rowsum_sh.at[idx2.at[0]], add=True)

    plsc.subcore_barrier()
    pltpu.sync_copy(rowsum_sh, rs_loc)

    _load_rc(w_own, 0, 0, 0)
    pltpu.async_copy(adj4.at[w_own, 0], adj_b.at[pl.ds(0, KCA), :], asem.at[0])

    @pl.loop(0, NCBA)
    def _(cb):
        p = lax.bitwise_and(cb, 1)
        band = pl.ds(p * KCA, KCA)
        _wait_rc(p, p)
        pltpu.make_async_copy(adj4.at[0, 0], adj_b.at[band, :],
                              asem.at[p]).wait()

        @pl.when(cb + 1 < NCBA)
        def _():
            nband = pl.ds((1 - p) * KCA, KCA)
            _load_rc(w_own, cb + 1, 1 - p, 1 - p)
            pltpu.async_copy(adj4.at[w_own, cb + 1], adj_b.at[nband, :],
                             asem.at[1 - p])

        @pl.when(cb >= 2)
        def _():
            pltpu.make_async_copy(vals_b.at[band, :], vals4.at[w_own, 0],
                                  wsem.at[p]).wait()

        @pl.loop(0, KCA)
        def _(k):
            kk = p * KCA + k
            for j in range(C // L):
                r = row_b[kk, pl.ds(j * L, L)]
                cc = col_b[kk, pl.ds(j * L, L)]
                e = jnp.exp(plsc.load_gather(a1b, [r]) +
                            plsc.load_gather(a2b, [cc]))
                rs = plsc.load_gather(rs_loc, _split(r))
                av = adj_b[kk, pl.ds(j * L, L)]
                vals_b[kk, pl.ds(j * L, L)] = (
                    (e / (rs + 1e-6) + 0.5 * av) * (1.0 / 1.5))

        pltpu.async_copy(vals_b.at[band, :], vals4.at[w_own, cb], wsem.at[p])

    for p in range(2):
        pltpu.make_async_copy(vals_b.at[pl.ds(p * KCA, KCA), :],
                              vals4.at[w_own, 0], wsem.at[p]).wait()


def _scb_body(row4, col4, vals4, emb_hbm,
              partial_hbm,
              row_b, col_b, vals_b, rows3,
              rcsem, vsem, gsem, ssem,
              out_sh):
    s = lax.axis_index("s")
    c = lax.axis_index("c")
    w_own = c * NS + s

    zeros16 = jnp.zeros((L,), jnp.float32)

    @pl.loop(0, C)
    def _(i):
        for j in range(H // L):
            rows3[0, i, pl.ds(j * L, L)] = zeros16

    for k in range(RPT // C):
        pltpu.sync_copy(rows3.at[0], out_sh.at[pl.ds(s * RPT + k * C, C), :])

    plsc.subcore_barrier()

    def _load_rcv(cb, p):
        band = pl.ds(p * KC, KC)
        pltpu.async_copy(row4.at[w_own, cb], row_b.at[band, :], rcsem.at[p])
        pltpu.async_copy(col4.at[w_own, cb], col_b.at[band, :], rcsem.at[p])
        pltpu.async_copy(vals4.at[w_own, cb], vals_b.at[band, :], vsem.at[p])

    def _wait_rcv(p):
        band = pl.ds(p * KC, KC)
        pltpu.make_async_copy(row4.at[0, 0], row_b.at[band, :],
                              rcsem.at[p]).wait()
        pltpu.make_async_copy(col4.at[0, 0], col_b.at[band, :],
                              rcsem.at[p]).wait()
        pltpu.make_async_copy(vals4.at[0, 0], vals_b.at[band, :],
                              vsem.at[p]).wait()

    _load_rcv(0, 0)
    _wait_rcv(0)

    def _gather(t, q):
        cbp = lax.bitwise_and(lax.div(t, KC), 1)
        pltpu.async_copy(emb_hbm.at[col_b.at[cbp * KC + lax.rem(t, KC)]],
                         rows3.at[q], gsem.at[q])

    _gather(jnp.int32(0), jnp.int32(0))
    _gather(jnp.int32(1), jnp.int32(1))

    NT = EW // C

    @pl.loop(0, NT)
    def _(t):
        k = lax.rem(t, KC)
        cb = lax.div(t, KC)
        p = lax.bitwise_and(cb, 1)
        q = lax.rem(t, NBUF)
        kk = p * KC + k

        pltpu.make_async_copy(emb_hbm.at[col_b.at[0]],
                              rows3.at[q], gsem.at[q]).wait()

        for jb in range(C // L):
            v16 = vals_b[kk, pl.ds(jb * L, L)]
            for i in range(L):
                ri = jb * L + i
                v = v16[i]
                for j2 in range(H // L):
                    sl = pl.ds(j2 * L, L)
                    rows3[q, ri, sl] = rows3[q, ri, sl] * v

        pltpu.async_copy(rows3.at[q], out_sh.at[row_b.at[kk]],
                         ssem.at[q], add=True)

        @pl.when(t + 2 < NT)
        def _():
            qn = lax.rem(t + 2, NBUF)

            @pl.when(t >= 2)
            def _():
                pltpu.make_async_copy(rows3.at[qn], out_sh.at[row_b.at[0]],
                                      ssem.at[qn]).wait()

            @pl.when(lax.rem(t + 2, KC) == 0)
            def _():
                _wait_rcv(1 - p)

            _gather(t + 2, qn)

        @pl.when(jnp.logical_and(k == 1, cb + 1 < NCB))
        def _():
            _load_rcv(cb + 1, 1 - p)

    for q in range(NBUF):
        pltpu.make_async_copy(rows3.at[q], out_sh.at[row_b.at[0]],
                              ssem.at[q]).wait()

    plsc.subcore_barrier()
    pltpu.sync_copy(out_sh.at[pl.ds(s * RPT, RPT), :],
                    partial_hbm.at[c, pl.ds(s * RPT, RPT), :])


@functools.cache
def _make_sca_call():
    return pl.kernel(
        _sca_body,
        out_type=jax.ShapeDtypeStruct((NW, NCBA, KCA, C), jnp.float32),
        mesh=plsc.VectorSubcoreMesh(core_axis_name="c", subcore_axis_name="s",
                                    num_cores=NC, num_subcores=NS),
        compiler_params=pltpu.CompilerParams(needs_layout_passes=False),
        scratch_types=(
            pltpu.VMEM((N,), jnp.float32),
            pltpu.VMEM((N,), jnp.float32),
            pltpu.VMEM((RS_R, 128), jnp.float32),
            pltpu.VMEM((2 * KCA, C), jnp.int32),
            pltpu.VMEM((2 * KCA, C), jnp.int32),
            pltpu.VMEM((2 * KCA, C), jnp.float32),
            pltpu.VMEM((2 * KCA, C), jnp.float32),
            pltpu.VMEM((1, RS_R), jnp.int32),
            pltpu.SemaphoreType.DMA((2,)),
            pltpu.SemaphoreType.DMA((2,)),
            pltpu.SemaphoreType.DMA((2,)),
            pltpu.VMEM_SHARED((RS_R, 128), jnp.float32),
        ),
    )


@functools.cache
def _make_scb_call():
    return pl.kernel(
        _scb_body,
        out_type=jax.ShapeDtypeStruct((NC, NP2, H), jnp.float32),
        mesh=plsc.VectorSubcoreMesh(core_axis_name="c", subcore_axis_name="s",
                                    num_cores=NC, num_subcores=NS),
        compiler_params=pltpu.CompilerParams(needs_layout_passes=False),
        scratch_types=(
            pltpu.VMEM((2 * KC, C), jnp.int32),
            pltpu.VMEM((2 * KC, C), jnp.int32),
            pltpu.VMEM((2 * KC, C), jnp.float32),
            pltpu.VMEM((NBUF, C, H), jnp.float32),
            pltpu.SemaphoreType.DMA((2,)),
            pltpu.SemaphoreType.DMA((2,)),
            pltpu.SemaphoreType.DMA((NBUF,)),
            pltpu.SemaphoreType.DMA((NBUF,)),
            pltpu.VMEM_SHARED((NP2, H), jnp.float32),
        ),
    )


def _sum_body(p_ref, o_ref):
    o_ref[:, :] = p_ref[0] + p_ref[1]


def _sum_partials(partial):
    blk = 2000
    return pl.pallas_call(
        _sum_body,
        grid=(N // blk,),
        in_specs=[pl.BlockSpec((NC, blk, H), lambda i: (0, i, 0))],
        out_specs=pl.BlockSpec((blk, H), lambda i: (i, 0)),
        out_shape=jax.ShapeDtypeStruct((N, H), jnp.float32),
    )(partial)


@jax.jit
def kernel(edge_index, adj_values, embeds, W1, b1, W2, b2, Wa, ba):
    row = edge_index[0, :].astype(jnp.int32)
    col = edge_index[1, :].astype(jnp.int32)
    wa1 = Wa[:H, 0].reshape(1, H)
    wa2 = Wa[H:, 0].reshape(1, H)

    a1, a2 = _node_scores(embeds, W1, b1, W2, b2, wa1, wa2, ba)

    rowA = row.reshape(NW, NCBA, KCA, C)
    colA = col.reshape(NW, NCBA, KCA, C)
    adjA = adj_values.reshape(NW, NCBA, KCA, C)
    row4 = row.reshape(NW, NCB, KC, C)
    col4 = col.reshape(NW, NCB, KC, C)

    values = _make_sca_call()(rowA, colA, adjA, a1, a2)
    partial = _make_scb_call()(row4, col4,
                               values.reshape(NW, NCB, KC, C), embeds)
    out = _sum_partials(partial)
    return values.reshape(E), out
